# Initial kernel scaffold; baseline (speedup 1.0000x reference)
import functools

import jax
import jax.numpy as jnp
from jax import lax
from jax.experimental import pallas as pl
from jax.experimental.pallas import tpu as pltpu
from jax.experimental.pallas import tpu_sc as plsc

NC = 2
NS = 16
NW = NC * NS
L = 16


def _rsqrt16(deg):
    safe = jnp.maximum(deg, 1.0)
    bits = plsc.bitcast(safe, jnp.int32)
    g = plsc.bitcast(jnp.int32(0x5F3759DF) - (bits >> 1), jnp.float32)
    h = safe * 0.5
    for _ in range(3):
        g = g * (1.5 - h * g * g)
    return jnp.where(deg > 0.0, g, 0.0)


def _make_k1(n, d, e, npad, ch, rch):
    seg = npad // NS
    eh = e // NS
    nch = eh // ch
    nchy = n // rch
    mesh = plsc.VectorSubcoreMesh(
        core_axis_name="c", subcore_axis_name="s", num_cores=NC, num_subcores=NS
    )

    @functools.partial(
        pl.kernel,
        out_type=(
            jax.ShapeDtypeStruct((n, d), jnp.float32),
            jax.ShapeDtypeStruct((npad,), jnp.float32),
        ),
        mesh=mesh,
        scratch_types=(
            pltpu.VMEM((ch,), jnp.int32),
            pltpu.VMEM((npad,), jnp.float32),
            pltpu.VMEM((NS, seg), jnp.float32),
            pltpu.VMEM((npad,), jnp.float32),
            pltpu.VMEM((seg,), jnp.float32),
            pltpu.VMEM((rch, d), jnp.float32),
            pltpu.VMEM_SHARED((NS, npad), jnp.float32),
            pltpu.VMEM_SHARED((npad,), jnp.float32),
        ),
    )
    def k1(x_hbm, src_hbm, tgt_hbm, y_hbm, dinvt_hbm,
           idx_v, hist_v, red_v, dinv_v, dvt_v, xbuf_v, sh_hist, sh_dinv):
        cid = lax.axis_index("c")
        sid = lax.axis_index("s")
        wid = cid * NS + sid
        ones = jnp.ones((L,), jnp.float32)
        zeros = jnp.zeros((L,), jnp.float32)

        def zero_hist(i, _):
            hist_v[pl.ds(i * L, L)] = zeros
            return 0

        def accumulate(idx_ref):
            def chunk(ci, _):
                pltpu.sync_copy(idx_ref.at[pl.ds(sid * eh + ci * ch, ch)], idx_v)

                def grp(g, _):
                    idx16 = idx_v[pl.ds(g * L, L)]
                    plsc.addupdate_scatter(hist_v, [idx16], ones)
                    return 0

                lax.fori_loop(0, ch // L, grp, 0)
                return 0

            lax.fori_loop(0, nch, chunk, 0)

        def reduce_dinv(out_v):
            for k in range(NS):
                pltpu.sync_copy(sh_hist.at[k, pl.ds(sid * seg, seg)], red_v.at[k])

            def red(i, _):
                acc = red_v[0, pl.ds(i * L, L)]
                for k in range(1, NS):
                    acc = acc + red_v[k, pl.ds(i * L, L)]
                out_v[pl.ds(i * L, L)] = _rsqrt16(acc)
                return 0

            lax.fori_loop(0, seg // L, red, 0)

        lax.fori_loop(0, npad // L, zero_hist, 0)
        accumulate(src_hbm)
        pltpu.sync_copy(hist_v, sh_hist.at[sid])
        plsc.subcore_barrier()
        reduce_dinv(dvt_v)
        pltpu.sync_copy(dvt_v, sh_dinv.at[pl.ds(sid * seg, seg)])
        plsc.subcore_barrier()
        pltpu.sync_copy(sh_dinv, dinv_v)

        def ychunk(jj, _):
            j = jj * NW + wid

            @pl.when(j < nchy)
            def _():
                row0 = j * rch
                pltpu.sync_copy(x_hbm.at[pl.ds(row0, rch), :], xbuf_v)

                def row(r, _):
                    wv = plsc.load_gather(
                        dinv_v, [jnp.full((L,), row0 + r, jnp.int32)]
                    )
                    for c in range(d // L):
                        sl = pl.ds(c * L, L)
                        xbuf_v[r, sl] = xbuf_v[r, sl] * wv
                    return 0

                lax.fori_loop(0, rch, row, 0)
                pltpu.sync_copy(xbuf_v, y_hbm.at[pl.ds(row0, rch), :])

            return 0

        lax.fori_loop(0, (nchy + NW - 1) // NW, ychunk, 0)

        lax.fori_loop(0, npad // L, zero_hist, 0)
        accumulate(tgt_hbm)
        plsc.subcore_barrier()
        pltpu.sync_copy(hist_v, sh_hist.at[sid])
        plsc.subcore_barrier()
        reduce_dinv(dvt_v)

        @pl.when(cid == 0)
        def _():
            pltpu.sync_copy(dvt_v, dinvt_hbm.at[pl.ds(sid * seg, seg)])

    return k1


def _make_k2(n, d, e, ce, zr):
    ept = e // NW
    rpt = n // NS
    mesh = plsc.VectorSubcoreMesh(
        core_axis_name="c", subcore_axis_name="s", num_cores=NC, num_subcores=NS
    )

    @functools.partial(
        pl.kernel,
        out_type=jax.ShapeDtypeStruct((NC, n, d), jnp.float32),
        mesh=mesh,
        scratch_types=(
            pltpu.VMEM((ce,), jnp.int32),
            pltpu.VMEM((ce,), jnp.int32),
            pltpu.VMEM((ce, d), jnp.float32),
            pltpu.VMEM((zr, d), jnp.float32),
            pltpu.VMEM_SHARED((n, d), jnp.float32),
            pltpu.SemaphoreType.DMA,
        ),
    )
    def k2(y_hbm, src_hbm, tgt_hbm, p_hbm, sidx_v, tidx_v, rows_v, zbuf,
           acc_sh, sem):
        cid = lax.axis_index("c")
        sid = lax.axis_index("s")
        wid = cid * NS + sid
        zeros = jnp.zeros((L,), jnp.float32)

        def zrow(r, _):
            for c in range(d // L):
                zbuf[r, pl.ds(c * L, L)] = zeros
            return 0

        lax.fori_loop(0, zr, zrow, 0)
        for k in range(rpt // zr):
            pltpu.sync_copy(zbuf, acc_sh.at[pl.ds(sid * rpt + k * zr, zr), :])
        plsc.subcore_barrier()

        base = wid * ept

        def chunk(i, _):
            off = base + i * ce
            pltpu.sync_copy(src_hbm.at[pl.ds(off, ce)], sidx_v)
            pltpu.sync_copy(tgt_hbm.at[pl.ds(off, ce)], tidx_v)
            pltpu.async_copy(y_hbm.at[sidx_v], rows_v, sem).wait()
            pltpu.sync_copy(rows_v, acc_sh.at[tidx_v], add=True)
            return 0

        lax.fori_loop(0, ept // ce, chunk, 0)
        plsc.subcore_barrier()
        pltpu.sync_copy(
            acc_sh.at[pl.ds(sid * rpt, rpt), :],
            p_hbm.at[cid, pl.ds(sid * rpt, rpt), :],
        )

    return k2


def _k3_body(p0_ref, p1_ref, dv_ref, o_ref):
    o_ref[...] = (p0_ref[...] + p1_ref[...]) * dv_ref[...]


def kernel(x, edge_index, num_nodes_target):
    n, d = x.shape
    e = edge_index.shape[1]
    npad = ((n + NS * L - 1) // (NS * L)) * (NS * L)
    src = edge_index[0]
    tgt = edge_index[1]

    y, dinvt = _make_k1(n, d, e, npad, ch=2000, rch=50)(x, src, tgt)
    p = _make_k2(n, d, e, ce=80, zr=125)(y, src, tgt)

    br = 400
    out = pl.pallas_call(
        _k3_body,
        grid=(n // br,),
        in_specs=[
            pl.BlockSpec((br, d), lambda i: (i, 0)),
            pl.BlockSpec((br, d), lambda i: (i, 0)),
            pl.BlockSpec((br, 1), lambda i: (i, 0)),
        ],
        out_specs=pl.BlockSpec((br, d), lambda i: (i, 0)),
        out_shape=jax.ShapeDtypeStruct((n, d), jnp.float32),
    )(p[0], p[1], dinvt[:n].reshape(n, 1))
    return out


# speedup vs baseline: 29.9952x; 29.9952x over previous
import functools

import jax
import jax.numpy as jnp
from jax import lax
from jax.experimental import pallas as pl
from jax.experimental.pallas import tpu as pltpu
from jax.experimental.pallas import tpu_sc as plsc

NC = 2
NS = 16
NW = NC * NS
L = 16


def _rsqrt16(deg):
    safe = jnp.maximum(deg, 1.0)
    bits = plsc.bitcast(safe, jnp.int32)
    g = plsc.bitcast(jnp.int32(0x5F3759DF) - (bits >> 1), jnp.float32)
    h = safe * 0.5
    for _ in range(3):
        g = g * (1.5 - h * g * g)
    return jnp.where(deg > 0.0, g, 0.0)


def _make_main(n, d, e, npad, ce, rch, hch, zr, nb=5):
    seg = npad // NS
    eh = e // NS
    nhc = eh // hch
    ept = e // NW
    nck = ept // ce
    nslot = nck + nb - 1
    ng = (nslot + nb - 1) // nb
    nyc = n // rch
    nypt = (nyc + NS - 1) // NS
    nzc = n // zr
    nzpt = (nzc + NS - 1) // NS
    ndc = n // rch
    ndpt = (ndc + NS - 1) // NS
    niq = npad // 128
    mesh = plsc.VectorSubcoreMesh(
        core_axis_name="c", subcore_axis_name="s", num_cores=NC, num_subcores=NS
    )

    scratch = (
        [pltpu.VMEM((ce,), jnp.int32) for _ in range(nb)]
        + [pltpu.VMEM((ce,), jnp.int32) for _ in range(nb)]
        + [pltpu.VMEM((ce, d), jnp.float32) for _ in range(nb)]
        + [
            pltpu.VMEM((hch,), jnp.int32),
            pltpu.VMEM((hch,), jnp.int32),
            pltpu.VMEM((npad,), jnp.float32),
            pltpu.VMEM((rch,), jnp.float32),
            pltpu.VMEM((rch,), jnp.float32),
            pltpu.VMEM((128,), jnp.int32),
            pltpu.VMEM((128,), jnp.int32),
            pltpu.VMEM((zr, d), jnp.float32),
            pltpu.VMEM((seg,), jnp.float32),
            pltpu.VMEM_SHARED((n, d), jnp.float32),
            pltpu.VMEM_SHARED((npad,), jnp.float32),
            pltpu.VMEM_SHARED((npad,), jnp.float32),
            pltpu.SemaphoreType.DMA((nb,)),
            pltpu.SemaphoreType.DMA((nb,)),
            pltpu.SemaphoreType.DMA((nb,)),
            pltpu.SemaphoreType.DMA((2,)),
            pltpu.SemaphoreType.DMA,
            pltpu.SemaphoreType.DMA((2,)),
        ]
    )

    @functools.partial(
        pl.kernel,
        out_type=(
            jax.ShapeDtypeStruct((n, d), jnp.float32),
            jax.ShapeDtypeStruct((n, d), jnp.float32),
            jax.ShapeDtypeStruct((NC, npad), jnp.float32),
            jax.ShapeDtypeStruct((n, d), jnp.float32),
            jax.ShapeDtypeStruct((n, d), jnp.float32),
        ),
        mesh=mesh,
        compiler_params=pltpu.CompilerParams(needs_layout_passes=False),
        scratch_types=tuple(scratch),
    )
    def km(x_hbm, src_hbm, tgt_hbm, z_hbm,
           p0_hbm, p1_hbm, ht_hbm, y0_hbm, y1_hbm, *scr):
        sidx = scr[:nb]
        tidx = scr[nb:2 * nb]
        rows = scr[2 * nb:3 * nb]
        (hb0, hb1, hist_v, degbuf, dinvbuf, iidx0, iidx1, zbuf, zseg,
         acc_sh, shs, sht, sem_i, sem_g, sem_s, sem_h, sem_z, sem_a) = \
            scr[3 * nb:]
        cid = lax.axis_index("c")
        sid = lax.axis_index("s")
        wid = cid * NS + sid
        ones = jnp.ones((L,), jnp.float32)
        zeros = jnp.zeros((L,), jnp.float32)
        iota = lax.iota(jnp.int32, L)
        tailmask = iota >= (L - ce % L) if ce % L else None
        hbufs = (hb0, hb1)
        iidxs = (iidx0, iidx1)

        def zs(i, _):
            zseg[pl.ds(i * L, L)] = zeros
            return 0

        lax.fori_loop(0, seg // L, zs, 0)

        def zh(i, _):
            hist_v[pl.ds(i * L, L)] = zeros
            return 0

        lax.fori_loop(0, npad // L, zh, 0)
        pltpu.sync_copy(z_hbm, zbuf)

        def zfire(jj, _):
            j = jj * NS + sid

            @pl.when(j < nzc)
            def _():
                pltpu.async_copy(zbuf, acc_sh.at[pl.ds(j * zr, zr), :], sem_z)

            return 0

        lax.fori_loop(0, nzpt, zfire, 0)
        pltpu.sync_copy(zseg, shs.at[pl.ds(sid * seg, seg)])
        pltpu.sync_copy(zseg, sht.at[pl.ds(sid * seg, seg)])

        pltpu.async_copy(
            src_hbm.at[pl.ds(sid * eh, hch)], hb0, sem_h.at[0]
        )
        for ci in range(nhc):
            b = ci % 2
            if ci + 1 < nhc:
                pltpu.async_copy(
                    src_hbm.at[pl.ds(sid * eh + (ci + 1) * hch, hch)],
                    hbufs[1 - b],
                    sem_h.at[1 - b],
                )
            pltpu.make_async_copy(
                src_hbm.at[pl.ds(sid * eh, hch)], hbufs[b], sem_h.at[b]
            ).wait()

            def grp(g, _):
                plsc.addupdate_scatter(
                    hist_v, [hbufs[b][pl.ds(g * L, L)]], ones
                )
                return 0

            lax.fori_loop(0, hch // L, grp, 0)

        plsc.subcore_barrier()

        def identity_add(dst_sh):
            def pair(q2, _):
                for par in range(2):
                    base = (q2 * 2 + par) * 128
                    for g2 in range(8):
                        iidxs[par][pl.ds(g2 * L, L)] = (
                            iota + (base + g2 * L)
                        )
                    pltpu.async_copy(
                        hist_v.at[pl.ds(base, 128)],
                        dst_sh.at[iidxs[par]],
                        sem_a.at[par],
                        add=True,
                    )
                for par in range(2):
                    base = (q2 * 2 + par) * 128
                    pltpu.make_async_copy(
                        hist_v.at[pl.ds(base, 128)],
                        dst_sh.at[iidxs[par]],
                        sem_a.at[par],
                    ).wait()
                return 0

            lax.fori_loop(0, niq // 2, pair, 0)

        identity_add(shs)
        plsc.subcore_barrier()

        lax.fori_loop(0, npad // L, zh, 0)

        def percore(y_hbm, p_hbm):
            def y_valid(jj):
                return (jj * NS + sid) < nyc

            def y_wait_out(jj, b):
                j = jj * NS + sid

                @pl.when(y_valid(jj))
                def _():
                    pltpu.make_async_copy(
                        rows[b], y_hbm.at[pl.ds(j * rch, rch), :], sem_s.at[b]
                    ).wait()

            def y_in(jj, b):
                j = jj * NS + sid

                @pl.when(y_valid(jj))
                def _():
                    pltpu.async_copy(
                        x_hbm.at[pl.ds(j * rch, rch), :], rows[b], sem_g.at[b]
                    )

            def y_process(jj, b):
                j = jj * NS + sid

                @pl.when(y_valid(jj))
                def _():
                    pltpu.make_async_copy(
                        x_hbm.at[pl.ds(j * rch, rch), :], rows[b], sem_g.at[b]
                    ).wait()
                    pltpu.sync_copy(shs.at[pl.ds(j * rch, rch)], degbuf)
                    for o in (0, L, rch - L):
                        dinvbuf[pl.ds(o, L)] = _rsqrt16(degbuf[pl.ds(o, L)])

                    def row(r, _):
                        rv = jnp.full((L,), r, jnp.int32)
                        wv = plsc.load_gather(
                            dinvbuf, [jnp.full((L,), r, jnp.int32)]
                        )
                        for c in range(d // L):
                            cv = iota + (c * L)
                            v = plsc.load_gather(rows[b], [rv, cv])
                            plsc.store_scatter(rows[b], [rv, cv], v * wv)
                        return 0

                    lax.fori_loop(0, rch, row, 0)
                    pltpu.async_copy(
                        rows[b], y_hbm.at[pl.ds(j * rch, rch), :], sem_s.at[b]
                    )

            y_in(0, 0)
            for jj in range(1, nypt):
                b = jj % nb
                if jj >= nb:
                    y_wait_out(jj - nb, b)
                y_in(jj, b)
                y_process(jj - 1, (jj - 1) % nb)
            y_process(nypt - 1, (nypt - 1) % nb)
            for jj in range(max(0, nypt - nb), nypt):
                y_wait_out(jj, jj % nb)

            def zdrain(jj, _):
                j = jj * NS + sid

                @pl.when(j < nzc)
                def _():
                    pltpu.make_async_copy(
                        zbuf, acc_sh.at[pl.ds(j * zr, zr), :], sem_z
                    ).wait()

                return 0

            lax.fori_loop(0, nzpt, zdrain, 0)
            plsc.subcore_barrier()

            base = wid * ept

            def slot(i, b):
                @pl.when((i >= nb) & (i < nck + nb))
                def _():
                    pltpu.make_async_copy(
                        rows[b], acc_sh.at[tidx[b]], sem_s.at[b]
                    ).wait()

                @pl.when(i < nck)
                def _():
                    off = base + i * ce
                    pltpu.async_copy(
                        src_hbm.at[pl.ds(off, ce)], sidx[b], sem_i.at[b]
                    )
                    pltpu.async_copy(
                        tgt_hbm.at[pl.ds(off, ce)], tidx[b], sem_i.at[b]
                    )

                j = i - 2
                bj = (b - 2) % nb

                @pl.when((j >= 0) & (j < nck))
                def _():
                    pltpu.make_async_copy(
                        src_hbm.at[pl.ds(0, ce)], sidx[bj], sem_i.at[bj]
                    ).wait()
                    pltpu.make_async_copy(
                        tgt_hbm.at[pl.ds(0, ce)], tidx[bj], sem_i.at[bj]
                    ).wait()
                    pltpu.async_copy(
                        y_hbm.at[sidx[bj]], rows[bj], sem_g.at[bj]
                    )

                k = i - 4
                bk = (b - 4) % nb

                @pl.when((k >= 0) & (k < nck))
                def _():
                    pltpu.make_async_copy(
                        y_hbm.at[sidx[bk]], rows[bk], sem_g.at[bk]
                    ).wait()
                    pltpu.async_copy(
                        rows[bk], acc_sh.at[tidx[bk]], sem_s.at[bk], add=True
                    )
                    for g in range(ce // L):
                        plsc.addupdate_scatter(
                            hist_v, [tidx[bk][pl.ds(g * L, L)]], ones
                        )
                    if ce % L:
                        plsc.addupdate_scatter(
                            hist_v,
                            [tidx[bk][pl.ds(ce - L, L)]],
                            ones,
                            mask=tailmask,
                        )

            def outer(g, _):
                for b in range(nb):
                    slot(g * nb + b, b)
                return 0

            lax.fori_loop(0, ng, outer, 0)
            plsc.subcore_barrier()

            def dfire(jj, _):
                j = jj * NS + sid

                @pl.when(j < ndc)
                def _():
                    pltpu.async_copy(
                        acc_sh.at[pl.ds(j * rch, rch), :],
                        p_hbm.at[pl.ds(j * rch, rch), :],
                        sem_z,
                    )

                return 0

            lax.fori_loop(0, ndpt, dfire, 0)

            def ddrain(jj, _):
                j = jj * NS + sid

                @pl.when(j < ndc)
                def _():
                    pltpu.make_async_copy(
                        acc_sh.at[pl.ds(j * rch, rch), :],
                        p_hbm.at[pl.ds(j * rch, rch), :],
                        sem_z,
                    ).wait()

                return 0

            lax.fori_loop(0, ndpt, ddrain, 0)

        @pl.when(cid == 0)
        def _():
            percore(y0_hbm, p0_hbm)

        @pl.when(cid == 1)
        def _():
            percore(y1_hbm, p1_hbm)

        identity_add(sht)
        plsc.subcore_barrier()
        pltpu.sync_copy(
            sht.at[pl.ds(sid * seg, seg)],
            ht_hbm.at[cid, pl.ds(sid * seg, seg)],
        )

    return km


def _k3_body(p0_ref, p1_ref, h0_ref, h1_ref, o_ref):
    deg = h0_ref[...] + h1_ref[...]
    dinv = jnp.where(deg > 0.0, jax.lax.rsqrt(deg), 0.0)
    o_ref[...] = (p0_ref[...] + p1_ref[...]) * dinv


def kernel(x, edge_index, num_nodes_target):
    n, d = x.shape
    e = edge_index.shape[1]
    npad = ((n + NS * L - 1) // (NS * L)) * (NS * L)
    src = edge_index[0]
    tgt = edge_index[1]

    zeros2d = jnp.zeros((8, d), jnp.float32)
    p0, p1, htp, _, _ = _make_main(
        n, d, e, npad, ce=40, rch=40, hch=2000, zr=8
    )(x, src, tgt, zeros2d)

    br = 400
    out = pl.pallas_call(
        _k3_body,
        grid=(n // br,),
        in_specs=[
            pl.BlockSpec((br, d), lambda i: (i, 0)),
            pl.BlockSpec((br, d), lambda i: (i, 0)),
            pl.BlockSpec((br, 1), lambda i: (i, 0)),
            pl.BlockSpec((br, 1), lambda i: (i, 0)),
        ],
        out_specs=pl.BlockSpec((br, d), lambda i: (i, 0)),
        out_shape=jax.ShapeDtypeStruct((n, d), jnp.float32),
    )(p0, p1, htp[0].reshape(npad, 1), htp[1].reshape(npad, 1))
    return out


# speedup vs baseline: 32.5925x; 1.0866x over previous
import functools

import jax
import jax.numpy as jnp
from jax import lax
from jax.experimental import pallas as pl
from jax.experimental.pallas import tpu as pltpu
from jax.experimental.pallas import tpu_sc as plsc

NC = 2
NS = 16
NW = NC * NS
L = 16


def _rsqrt16(deg):
    safe = jnp.maximum(deg, 1.0)
    bits = plsc.bitcast(safe, jnp.int32)
    g = plsc.bitcast(jnp.int32(0x5F3759DF) - (bits >> 1), jnp.float32)
    h = safe * 0.5
    for _ in range(3):
        g = g * (1.5 - h * g * g)
    return jnp.where(deg > 0.0, g, 0.0)


def _make_k1(n, d, e, npad, rch):
    seg = npad // NS
    eh = e // NS
    nchy = n // rch
    njj = (nchy + NW - 1) // NW
    mesh = plsc.VectorSubcoreMesh(
        core_axis_name="c", subcore_axis_name="s", num_cores=NC, num_subcores=NS
    )

    @functools.partial(
        pl.kernel,
        out_type=jax.ShapeDtypeStruct((n, d), jnp.float32),
        mesh=mesh,
        compiler_params=pltpu.CompilerParams(needs_layout_passes=False),
        scratch_types=(
            pltpu.VMEM((eh,), jnp.int32),
            pltpu.VMEM((npad,), jnp.float32),
            pltpu.VMEM((NS * seg,), jnp.float32),
            pltpu.VMEM((npad,), jnp.float32),
            pltpu.VMEM((seg,), jnp.float32),
            pltpu.VMEM((rch, d), jnp.float32),
            pltpu.VMEM((rch, d), jnp.float32),
            pltpu.VMEM_SHARED((NS * npad,), jnp.float32),
            pltpu.VMEM_SHARED((npad,), jnp.float32),
            pltpu.SemaphoreType.DMA,
            pltpu.SemaphoreType.DMA((2,)),
            pltpu.SemaphoreType.DMA((2,)),
        ),
    )
    def k1(x_hbm, src_hbm, y_hbm,
           sidx_all, hist_v, red_v, dinv_v, dvt_v, xbuf0, xbuf1,
           sh_hist, sh_dinv, sem_a, sem_yi, sem_yo):
        cid = lax.axis_index("c")
        sid = lax.axis_index("s")
        wid = cid * NS + sid
        ones = jnp.ones((L,), jnp.float32)
        zeros = jnp.zeros((L,), jnp.float32)
        xbufs = (xbuf0, xbuf1)

        pltpu.async_copy(src_hbm.at[pl.ds(sid * eh, eh)], sidx_all, sem_a)

        def zero_hist(i, _):
            hist_v[pl.ds(i * L, L)] = zeros
            return 0

        def accumulate(idx_all):
            def grp(g, _):
                idx16 = idx_all[pl.ds(g * L, L)]
                plsc.addupdate_scatter(hist_v, [idx16], ones)
                return 0

            lax.fori_loop(0, eh // L, grp, 0)

        def reduce_dinv(out_v):
            for k in range(NS):
                pltpu.async_copy(
                    sh_hist.at[pl.ds(k * npad + sid * seg, seg)],
                    red_v.at[pl.ds(k * seg, seg)],
                    sem_a,
                )
            for k in range(NS):
                pltpu.make_async_copy(
                    sh_hist.at[pl.ds(k * npad + sid * seg, seg)],
                    red_v.at[pl.ds(k * seg, seg)],
                    sem_a,
                ).wait()

            def red(i, _):
                acc = red_v[pl.ds(i * L, L)]
                for k in range(1, NS):
                    acc = acc + red_v[pl.ds(k * seg + i * L, L)]
                out_v[pl.ds(i * L, L)] = _rsqrt16(acc)
                return 0

            lax.fori_loop(0, seg // L, red, 0)

        lax.fori_loop(0, npad // L, zero_hist, 0)
        pltpu.make_async_copy(
            src_hbm.at[pl.ds(sid * eh, eh)], sidx_all, sem_a
        ).wait()
        accumulate(sidx_all)
        pltpu.sync_copy(hist_v, sh_hist.at[pl.ds(sid * npad, npad)])
        plsc.subcore_barrier()
        reduce_dinv(dvt_v)
        pltpu.sync_copy(dvt_v, sh_dinv.at[pl.ds(sid * seg, seg)])
        plsc.subcore_barrier()
        pltpu.sync_copy(sh_dinv, dinv_v)

        def y_in(jj, b):
            j = jj * NW + wid

            @pl.when(j < nchy)
            def _():
                pltpu.async_copy(
                    x_hbm.at[pl.ds(j * rch, rch), :], xbufs[b], sem_yi.at[b]
                )

        def y_wait_out(jj, b):
            j = jj * NW + wid

            @pl.when(j < nchy)
            def _():
                pltpu.make_async_copy(
                    xbufs[b], y_hbm.at[pl.ds(j * rch, rch), :], sem_yo.at[b]
                ).wait()

        y_in(0, 0)
        for jj in range(njj):
            b = jj % 2
            if jj >= 2:
                y_wait_out(jj - 2, b)
            if jj + 1 < njj:
                y_in(jj + 1, 1 - b)
            j = jj * NW + wid

            @pl.when(j < nchy)
            def _():
                row0 = j * rch
                pltpu.make_async_copy(
                    x_hbm.at[pl.ds(row0, rch), :], xbufs[b], sem_yi.at[b]
                ).wait()

                def row(r, _):
                    rv = jnp.full((L,), r, jnp.int32)
                    wv = plsc.load_gather(
                        dinv_v, [jnp.full((L,), row0 + r, jnp.int32)]
                    )
                    for c in range(d // L):
                        cv = lax.iota(jnp.int32, L) + (c * L)
                        v = plsc.load_gather(xbufs[b], [rv, cv])
                        plsc.store_scatter(xbufs[b], [rv, cv], v * wv)
                    return 0

                lax.fori_loop(0, rch, row, 0)
                pltpu.async_copy(
                    xbufs[b], y_hbm.at[pl.ds(row0, rch), :], sem_yo.at[b]
                )

        for jj in (njj - 2, njj - 1):
            if jj >= 0:
                y_wait_out(jj, jj % 2)

    return k1


def _make_k2(n, d, e, npad, ce, zr, nb=5):
    seg = npad // NS
    ept = e // NW
    nck = ept // ce
    nzc = n // zr
    nslot = nck + nb - 1
    ng = (nslot + nb - 1) // nb
    mesh = plsc.VectorSubcoreMesh(
        core_axis_name="c", subcore_axis_name="s", num_cores=NC, num_subcores=NS
    )

    scratch = (
        [pltpu.VMEM((ce,), jnp.int32) for _ in range(nb)]
        + [pltpu.VMEM((ce,), jnp.int32) for _ in range(nb)]
        + [pltpu.VMEM((ce, d), jnp.float32) for _ in range(nb)]
        + [
            pltpu.VMEM((zr, d), jnp.float32),
            pltpu.VMEM((npad,), jnp.float32),
            pltpu.VMEM((seg,), jnp.float32),
            pltpu.VMEM((128,), jnp.int32),
            pltpu.VMEM((128,), jnp.int32),
            pltpu.VMEM_SHARED((n, d), jnp.float32),
            pltpu.VMEM_SHARED((npad,), jnp.float32),
            pltpu.SemaphoreType.DMA((nb,)),
            pltpu.SemaphoreType.DMA((nb,)),
            pltpu.SemaphoreType.DMA((nb,)),
            pltpu.SemaphoreType.DMA((2,)),
        ]
    )

    @functools.partial(
        pl.kernel,
        out_type=(
            jax.ShapeDtypeStruct((n, d), jnp.float32),
            jax.ShapeDtypeStruct((n, d), jnp.float32),
            jax.ShapeDtypeStruct((npad,), jnp.float32),
            jax.ShapeDtypeStruct((npad,), jnp.float32),
        ),
        mesh=mesh,
        compiler_params=pltpu.CompilerParams(needs_layout_passes=False),
        scratch_types=tuple(scratch),
    )
    def k2(y_hbm, src_hbm, tgt_hbm, z_hbm, p0_hbm, p1_hbm, ht_hbm, *scr):
        sidx = scr[:nb]
        tidx = scr[nb:2 * nb]
        rows = scr[2 * nb:3 * nb]
        zbuf, hist_v, dvt_v, acc_sh, sh_hist, sem_i, sem_g, sem_s = scr[3 * nb:]
        cid = lax.axis_index("c")
        sid = lax.axis_index("s")
        wid = cid * NS + sid
        ones = jnp.ones((L,), jnp.float32)
        zeros = jnp.zeros((L,), jnp.float32)
        tailmask = lax.iota(jnp.int32, L) >= (2 * L - ce % L)
        pltpu.sync_copy(z_hbm, zbuf)

        def zero_hist(i, _):
            hist_v[pl.ds(i * L, L)] = zeros
            return 0

        lax.fori_loop(0, npad // L, zero_hist, 0)
        nzcpt = (nzc + NS - 1) // NS
        for jj in range(nzcpt):
            j = jj * NS + sid

            @pl.when(j < nzc)
            def _():
                pltpu.async_copy(
                    zbuf, acc_sh.at[pl.ds(j * zr, zr), :], sem_i.at[0]
                )
        for jj in range(nzcpt):
            j = jj * NS + sid

            @pl.when(j < nzc)
            def _():
                pltpu.make_async_copy(
                    zbuf, acc_sh.at[pl.ds(j * zr, zr), :], sem_i.at[0]
                ).wait()
        plsc.subcore_barrier()

        base = wid * ept

        def slot(i, b):
            @pl.when((i >= nb) & (i < nck + nb))
            def _():
                pltpu.make_async_copy(
                    rows[b], acc_sh.at[tidx[b]], sem_s.at[b]
                ).wait()

            @pl.when(i < nck)
            def _():
                off = base + i * ce
                pltpu.async_copy(src_hbm.at[pl.ds(off, ce)], sidx[b], sem_i.at[b])
                pltpu.async_copy(tgt_hbm.at[pl.ds(off, ce)], tidx[b], sem_i.at[b])

            j = i - 2
            bj = (b - 2) % nb

            @pl.when((j >= 0) & (j < nck))
            def _():
                pltpu.make_async_copy(
                    src_hbm.at[pl.ds(0, ce)], sidx[bj], sem_i.at[bj]
                ).wait()
                pltpu.make_async_copy(
                    tgt_hbm.at[pl.ds(0, ce)], tidx[bj], sem_i.at[bj]
                ).wait()
                pltpu.async_copy(y_hbm.at[sidx[bj]], rows[bj], sem_g.at[bj])

            k = i - 4
            bk = (b - 4) % nb

            @pl.when((k >= 0) & (k < nck))
            def _():
                pltpu.make_async_copy(
                    y_hbm.at[sidx[bk]], rows[bk], sem_g.at[bk]
                ).wait()
                pltpu.async_copy(
                    rows[bk], acc_sh.at[tidx[bk]], sem_s.at[bk], add=True
                )
                for g in range(ce // L):
                    plsc.addupdate_scatter(
                        hist_v, [tidx[bk][pl.ds(g * L, L)]], ones
                    )
                if ce % L:
                    plsc.addupdate_scatter(
                        hist_v,
                        [tidx[bk][pl.ds(ce - L, L)]],
                        ones,
                        mask=tailmask,
                    )

        def outer(g, _):
            for b in range(nb):
                slot(g * nb + b, b)
            return 0

        lax.fori_loop(0, ng, outer, 0)
        plsc.subcore_barrier()

        def pair(q2, _):
            for par in range(2):
                base = (q2 * 2 + par) * 128
                for g2 in range(8):
                    iidxs[par][pl.ds(g2 * L, L)] = iota + (base + g2 * L)
                pltpu.async_copy(
                    hist_v.at[pl.ds(base, 128)],
                    sht.at[iidxs[par]],
                    sem_a.at[par],
                    add=True,
                )
            for par in range(2):
                base = (q2 * 2 + par) * 128
                pltpu.make_async_copy(
                    hist_v.at[pl.ds(base, 128)],
                    sht.at[iidxs[par]],
                    sem_a.at[par],
                ).wait()
            return 0

        lax.fori_loop(0, npad // 256, pair, 0)
        plsc.subcore_barrier()

        @pl.when(cid == 0)
        def _():
            pltpu.sync_copy(
                sht.at[pl.ds(sid * seg, seg)],
                ht0_hbm.at[pl.ds(sid * seg, seg)],
            )

        @pl.when(cid == 1)
        def _():
            pltpu.sync_copy(
                sht.at[pl.ds(sid * seg, seg)],
                ht1_hbm.at[pl.ds(sid * seg, seg)],
            )

        def dump(p_hbm):
            for jj in range(nzcpt):
                j = jj * NS + sid

                @pl.when(j < nzc)
                def _():
                    pltpu.async_copy(
                        acc_sh.at[pl.ds(j * zr, zr), :],
                        p_hbm.at[pl.ds(j * zr, zr), :],
                        sem_i.at[0],
                    )
            for jj in range(nzcpt):
                j = jj * NS + sid

                @pl.when(j < nzc)
                def _():
                    pltpu.make_async_copy(
                        acc_sh.at[pl.ds(j * zr, zr), :],
                        p_hbm.at[pl.ds(j * zr, zr), :],
                        sem_i.at[0],
                    ).wait()

        @pl.when(cid == 0)
        def _():
            dump(p0_hbm)

        @pl.when(cid == 1)
        def _():
            dump(p1_hbm)

    return k2


def _k3_body(p0_ref, p1_ref, h0_ref, h1_ref, o_ref):
    deg = h0_ref[...] + h1_ref[...]
    dinv = jnp.where(deg > 0.0, jax.lax.rsqrt(deg), 0.0)
    o_ref[...] = (p0_ref[...] + p1_ref[...]) * dinv


def kernel(x, edge_index, num_nodes_target):
    n, d = x.shape
    e = edge_index.shape[1]
    npad = ((n + NS * L - 1) // (NS * L)) * (NS * L)
    src = edge_index[0]
    tgt = edge_index[1]

    y = _make_k1(n, d, e, npad, rch=80)(x, src)
    zeros2d = jnp.zeros((80, d), jnp.float32)
    p0, p1, ht0, ht1 = _make_k2(n, d, e, npad, ce=40, zr=80)(
        y, src, tgt, zeros2d
    )

    br = 400
    out = pl.pallas_call(
        _k3_body,
        grid=(n // br,),
        in_specs=[
            pl.BlockSpec((br, d), lambda i: (i, 0)),
            pl.BlockSpec((br, d), lambda i: (i, 0)),
            pl.BlockSpec((br, 1), lambda i: (i, 0)),
            pl.BlockSpec((br, 1), lambda i: (i, 0)),
        ],
        out_specs=pl.BlockSpec((br, d), lambda i: (i, 0)),
        out_shape=jax.ShapeDtypeStruct((n, d), jnp.float32),
    )(p0, p1, ht0.reshape(npad, 1), ht1.reshape(npad, 1))
    return out


# speedup vs baseline: 35.0940x; 1.0768x over previous
import functools

import jax
import jax.numpy as jnp
from jax import lax
from jax.experimental import pallas as pl
from jax.experimental.pallas import tpu as pltpu
from jax.experimental.pallas import tpu_sc as plsc

NC = 2
NS = 16
NW = NC * NS
L = 16


def _rsqrt16(deg):
    safe = jnp.maximum(deg, 1.0)
    bits = plsc.bitcast(safe, jnp.int32)
    g = plsc.bitcast(jnp.int32(0x5F3759DF) - (bits >> 1), jnp.float32)
    h = safe * 0.5
    for _ in range(3):
        g = g * (1.5 - h * g * g)
    return jnp.where(deg > 0.0, g, 0.0)


def _make_k1(n, d, e, npad, rch):
    seg = npad // NS
    eh = e // NS
    nchy = n // rch
    njj = (nchy + NW - 1) // NW
    mesh = plsc.VectorSubcoreMesh(
        core_axis_name="c", subcore_axis_name="s", num_cores=NC, num_subcores=NS
    )

    @functools.partial(
        pl.kernel,
        out_type=jax.ShapeDtypeStruct((n, d), jnp.float32),
        mesh=mesh,
        compiler_params=pltpu.CompilerParams(needs_layout_passes=False),
        scratch_types=(
            pltpu.VMEM((eh,), jnp.int32),
            pltpu.VMEM((npad,), jnp.float32),
            pltpu.VMEM((NS * seg,), jnp.float32),
            pltpu.VMEM((npad,), jnp.float32),
            pltpu.VMEM((seg,), jnp.float32),
            pltpu.VMEM((rch, d), jnp.float32),
            pltpu.VMEM((rch, d), jnp.float32),
            pltpu.VMEM_SHARED((NS * npad,), jnp.float32),
            pltpu.VMEM_SHARED((npad,), jnp.float32),
            pltpu.SemaphoreType.DMA,
            pltpu.SemaphoreType.DMA((2,)),
            pltpu.SemaphoreType.DMA((2,)),
        ),
    )
    def k1(x_hbm, src_hbm, y_hbm,
           sidx_all, hist_v, red_v, dinv_v, dvt_v, xbuf0, xbuf1,
           sh_hist, sh_dinv, sem_a, sem_yi, sem_yo):
        cid = lax.axis_index("c")
        sid = lax.axis_index("s")
        wid = cid * NS + sid
        ones = jnp.ones((L,), jnp.float32)
        zeros = jnp.zeros((L,), jnp.float32)
        xbufs = (xbuf0, xbuf1)

        pltpu.async_copy(src_hbm.at[pl.ds(sid * eh, eh)], sidx_all, sem_a)

        def zero_hist(i, _):
            hist_v[pl.ds(i * L, L)] = zeros
            return 0

        def accumulate(idx_all):
            def grp(g, _):
                idx16 = idx_all[pl.ds(g * L, L)]
                plsc.addupdate_scatter(hist_v, [idx16], ones)
                return 0

            lax.fori_loop(0, eh // L, grp, 0)

        def reduce_dinv(out_v):
            for k in range(NS):
                pltpu.async_copy(
                    sh_hist.at[pl.ds(k * npad + sid * seg, seg)],
                    red_v.at[pl.ds(k * seg, seg)],
                    sem_a,
                )
            for k in range(NS):
                pltpu.make_async_copy(
                    sh_hist.at[pl.ds(k * npad + sid * seg, seg)],
                    red_v.at[pl.ds(k * seg, seg)],
                    sem_a,
                ).wait()

            def red(i, _):
                acc = red_v[pl.ds(i * L, L)]
                for k in range(1, NS):
                    acc = acc + red_v[pl.ds(k * seg + i * L, L)]
                out_v[pl.ds(i * L, L)] = _rsqrt16(acc)
                return 0

            lax.fori_loop(0, seg // L, red, 0)

        lax.fori_loop(0, npad // L, zero_hist, 0)
        pltpu.make_async_copy(
            src_hbm.at[pl.ds(sid * eh, eh)], sidx_all, sem_a
        ).wait()
        accumulate(sidx_all)
        pltpu.sync_copy(hist_v, sh_hist.at[pl.ds(sid * npad, npad)])
        plsc.subcore_barrier()
        reduce_dinv(dvt_v)
        pltpu.sync_copy(dvt_v, sh_dinv.at[pl.ds(sid * seg, seg)])
        plsc.subcore_barrier()
        pltpu.sync_copy(sh_dinv, dinv_v)

        def y_in(jj, b):
            j = jj * NW + wid

            @pl.when(j < nchy)
            def _():
                pltpu.async_copy(
                    x_hbm.at[pl.ds(j * rch, rch), :], xbufs[b], sem_yi.at[b]
                )

        def y_wait_out(jj, b):
            j = jj * NW + wid

            @pl.when(j < nchy)
            def _():
                pltpu.make_async_copy(
                    xbufs[b], y_hbm.at[pl.ds(j * rch, rch), :], sem_yo.at[b]
                ).wait()

        y_in(0, 0)
        for jj in range(njj):
            b = jj % 2
            if jj >= 2:
                y_wait_out(jj - 2, b)
            if jj + 1 < njj:
                y_in(jj + 1, 1 - b)
            j = jj * NW + wid

            @pl.when(j < nchy)
            def _():
                row0 = j * rch
                pltpu.make_async_copy(
                    x_hbm.at[pl.ds(row0, rch), :], xbufs[b], sem_yi.at[b]
                ).wait()

                def row(r, _):
                    rv = jnp.full((L,), r, jnp.int32)
                    wv = plsc.load_gather(
                        dinv_v, [jnp.full((L,), row0 + r, jnp.int32)]
                    )
                    for c in range(d // L):
                        cv = lax.iota(jnp.int32, L) + (c * L)
                        v = plsc.load_gather(xbufs[b], [rv, cv])
                        plsc.store_scatter(xbufs[b], [rv, cv], v * wv)
                    return 0

                lax.fori_loop(0, rch, row, 0)
                pltpu.async_copy(
                    xbufs[b], y_hbm.at[pl.ds(row0, rch), :], sem_yo.at[b]
                )

        for jj in (njj - 2, njj - 1):
            if jj >= 0:
                y_wait_out(jj, jj % 2)

    return k1


def _make_k2(n, d, e, npad, ce, nb=4):
    seg = npad // NS
    ept = e // NW
    nck = ept // ce
    nzc = n // ce
    nzpt = (nzc + NS - 1) // NS
    nslot = nck + nb - 1
    ng = (nslot + nb - 1) // nb
    mesh = plsc.VectorSubcoreMesh(
        core_axis_name="c", subcore_axis_name="s", num_cores=NC, num_subcores=NS
    )

    scratch = (
        [pltpu.VMEM((ce,), jnp.int32) for _ in range(nb)]
        + [pltpu.VMEM((ce,), jnp.int32) for _ in range(nb)]
        + [pltpu.VMEM((ce, d), jnp.float32) for _ in range(nb)]
        + [
            pltpu.VMEM((ce,), jnp.float32),
            pltpu.VMEM((seg,), jnp.float32),
            pltpu.VMEM_SHARED((n, d), jnp.float32),
            pltpu.VMEM_SHARED((npad,), jnp.float32),
            pltpu.SemaphoreType.DMA((nb,)),
            pltpu.SemaphoreType.DMA((nb,)),
            pltpu.SemaphoreType.DMA((nb,)),
            pltpu.SemaphoreType.DMA((nb,)),
            pltpu.SemaphoreType.DMA,
        ]
    )

    @functools.partial(
        pl.kernel,
        out_type=(
            jax.ShapeDtypeStruct((n, d), jnp.float32),
            jax.ShapeDtypeStruct((n, d), jnp.float32),
            jax.ShapeDtypeStruct((npad,), jnp.float32),
            jax.ShapeDtypeStruct((npad,), jnp.float32),
        ),
        mesh=mesh,
        compiler_params=pltpu.CompilerParams(needs_layout_passes=False),
        scratch_types=tuple(scratch),
    )
    def k2(y_hbm, src_hbm, tgt_hbm, z_hbm, p0_hbm, p1_hbm, ht0_hbm, ht1_hbm,
           *scr):
        sidx = scr[:nb]
        tidx = scr[nb:2 * nb]
        rows = scr[2 * nb:3 * nb]
        (ones_v, zseg, acc_sh, sht,
         sem_i, sem_g, sem_s, sem_t, sem_z) = scr[3 * nb:]
        cid = lax.axis_index("c")
        sid = lax.axis_index("s")
        wid = cid * NS + sid
        ones = jnp.ones((L,), jnp.float32)
        zeros = jnp.zeros((L,), jnp.float32)

        def fill(i, _):
            ones_v[pl.ds(i * L, L)] = ones
            zseg[pl.ds(i * L, L)] = zeros
            return 0

        lax.fori_loop(0, seg // L, fill, 0)
        pltpu.sync_copy(zseg, sht.at[pl.ds(sid * seg, seg)])

        def zfire(jj, _):
            j = jj * NS + sid

            @pl.when(j < nzc)
            def _():
                pltpu.async_copy(
                    z_hbm, acc_sh.at[pl.ds(j * ce, ce), :], sem_z
                )

            return 0

        lax.fori_loop(0, nzpt, zfire, 0)

        def zdrain(jj, _):
            j = jj * NS + sid

            @pl.when(j < nzc)
            def _():
                pltpu.make_async_copy(
                    z_hbm, acc_sh.at[pl.ds(j * ce, ce), :], sem_z
                ).wait()

            return 0

        lax.fori_loop(0, nzpt, zdrain, 0)
        plsc.subcore_barrier()

        base = wid * ept

        def slot(i, b):
            @pl.when((i >= nb) & (i < nck + nb))
            def _():
                pltpu.make_async_copy(
                    rows[b], acc_sh.at[tidx[b]], sem_s.at[b]
                ).wait()
                pltpu.make_async_copy(
                    ones_v, sht.at[tidx[b]], sem_t.at[b]
                ).wait()

            @pl.when(i < nck)
            def _():
                off = base + i * ce
                pltpu.async_copy(
                    src_hbm.at[pl.ds(off, ce)], sidx[b], sem_i.at[b]
                )
                pltpu.async_copy(
                    tgt_hbm.at[pl.ds(off, ce)], tidx[b], sem_i.at[b]
                )

            j = i - 1
            bj = (b - 1) % nb

            @pl.when((j >= 0) & (j < nck))
            def _():
                pltpu.make_async_copy(
                    src_hbm.at[pl.ds(0, ce)], sidx[bj], sem_i.at[bj]
                ).wait()
                pltpu.make_async_copy(
                    tgt_hbm.at[pl.ds(0, ce)], tidx[bj], sem_i.at[bj]
                ).wait()
                pltpu.async_copy(
                    y_hbm.at[sidx[bj]], rows[bj], sem_g.at[bj]
                )

            k = i - 3
            bk = (b - 3) % nb

            @pl.when((k >= 0) & (k < nck))
            def _():
                pltpu.make_async_copy(
                    y_hbm.at[sidx[bk]], rows[bk], sem_g.at[bk]
                ).wait()
                pltpu.async_copy(
                    rows[bk], acc_sh.at[tidx[bk]], sem_s.at[bk], add=True
                )
                pltpu.async_copy(
                    ones_v, sht.at[tidx[bk]], sem_t.at[bk], add=True
                )

        def outer(g, _):
            for b in range(nb):
                slot(g * nb + b, b)
            return 0

        lax.fori_loop(0, ng, outer, 0)
        plsc.subcore_barrier()

        @pl.when(cid == 0)
        def _():
            pltpu.sync_copy(
                sht.at[pl.ds(sid * seg, seg)],
                ht0_hbm.at[pl.ds(sid * seg, seg)],
            )

        @pl.when(cid == 1)
        def _():
            pltpu.sync_copy(
                sht.at[pl.ds(sid * seg, seg)],
                ht1_hbm.at[pl.ds(sid * seg, seg)],
            )

        def dump(p_hbm):
            def dfire(jj, _):
                j = jj * NS + sid

                @pl.when(j < nzc)
                def _():
                    pltpu.async_copy(
                        acc_sh.at[pl.ds(j * ce, ce), :],
                        p_hbm.at[pl.ds(j * ce, ce), :],
                        sem_z,
                    )

                return 0

            lax.fori_loop(0, nzpt, dfire, 0)

            def ddrain(jj, _):
                j = jj * NS + sid

                @pl.when(j < nzc)
                def _():
                    pltpu.make_async_copy(
                        acc_sh.at[pl.ds(j * ce, ce), :],
                        p_hbm.at[pl.ds(j * ce, ce), :],
                        sem_z,
                    ).wait()

                return 0

            lax.fori_loop(0, nzpt, ddrain, 0)

        @pl.when(cid == 0)
        def _():
            dump(p0_hbm)

        @pl.when(cid == 1)
        def _():
            dump(p1_hbm)

    return k2


def _k3_body(p0_ref, p1_ref, h0_ref, h1_ref, o_ref):
    deg = h0_ref[...] + h1_ref[...]
    dinv = jnp.where(deg > 0.0, jax.lax.rsqrt(deg), 0.0)
    o_ref[...] = (p0_ref[...] + p1_ref[...]) * dinv


def kernel(x, edge_index, num_nodes_target):
    n, d = x.shape
    e = edge_index.shape[1]
    npad = ((n + NS * L - 1) // (NS * L)) * (NS * L)
    src = edge_index[0]
    tgt = edge_index[1]

    y = _make_k1(n, d, e, npad, rch=80)(x, src)
    zeros2d = jnp.zeros((80, d), jnp.float32)
    p0, p1, ht0, ht1 = _make_k2(n, d, e, npad, ce=80)(y, src, tgt, zeros2d)

    br = 400
    out = pl.pallas_call(
        _k3_body,
        grid=(n // br,),
        in_specs=[
            pl.BlockSpec((br, d), lambda i: (i, 0)),
            pl.BlockSpec((br, d), lambda i: (i, 0)),
            pl.BlockSpec((br, 1), lambda i: (i, 0)),
            pl.BlockSpec((br, 1), lambda i: (i, 0)),
        ],
        out_specs=pl.BlockSpec((br, d), lambda i: (i, 0)),
        out_shape=jax.ShapeDtypeStruct((n, d), jnp.float32),
    )(p0, p1, ht0.reshape(npad, 1), ht1.reshape(npad, 1))
    return out


# speedup vs baseline: 35.1129x; 1.0005x over previous
import functools

import jax
import jax.numpy as jnp
from jax import lax
from jax.experimental import pallas as pl
from jax.experimental.pallas import tpu as pltpu
from jax.experimental.pallas import tpu_sc as plsc

NC = 2
NS = 16
NW = NC * NS
L = 16


def _rsqrt16(deg):
    safe = jnp.maximum(deg, 1.0)
    bits = plsc.bitcast(safe, jnp.int32)
    g = plsc.bitcast(jnp.int32(0x5F3759DF) - (bits >> 1), jnp.float32)
    h = safe * 0.5
    for _ in range(3):
        g = g * (1.5 - h * g * g)
    return jnp.where(deg > 0.0, g, 0.0)


def _make_k1(n, d, e, npad, rch):
    seg = npad // NS
    eh = e // NS
    nchy = n // rch
    njj = (nchy + NW - 1) // NW
    mesh = plsc.VectorSubcoreMesh(
        core_axis_name="c", subcore_axis_name="s", num_cores=NC, num_subcores=NS
    )

    @functools.partial(
        pl.kernel,
        out_type=jax.ShapeDtypeStruct((n, d), jnp.float32),
        mesh=mesh,
        compiler_params=pltpu.CompilerParams(needs_layout_passes=False),
        scratch_types=(
            pltpu.VMEM((eh,), jnp.int32),
            pltpu.VMEM((npad,), jnp.float32),
            pltpu.VMEM((NS * seg,), jnp.float32),
            pltpu.VMEM((npad,), jnp.float32),
            pltpu.VMEM((seg,), jnp.float32),
            pltpu.VMEM((rch, d), jnp.float32),
            pltpu.VMEM((rch, d), jnp.float32),
            pltpu.VMEM_SHARED((NS * npad,), jnp.float32),
            pltpu.VMEM_SHARED((npad,), jnp.float32),
            pltpu.SemaphoreType.DMA,
            pltpu.SemaphoreType.DMA((2,)),
            pltpu.SemaphoreType.DMA((2,)),
        ),
    )
    def k1(x_hbm, src_hbm, y_hbm,
           sidx_all, hist_v, red_v, dinv_v, dvt_v, xbuf0, xbuf1,
           sh_hist, sh_dinv, sem_a, sem_yi, sem_yo):
        cid = lax.axis_index("c")
        sid = lax.axis_index("s")
        wid = cid * NS + sid
        ones = jnp.ones((L,), jnp.float32)
        zeros = jnp.zeros((L,), jnp.float32)
        xbufs = (xbuf0, xbuf1)

        pltpu.async_copy(src_hbm.at[pl.ds(sid * eh, eh)], sidx_all, sem_a)

        def zero_hist(i, _):
            hist_v[pl.ds(i * L, L)] = zeros
            return 0

        def accumulate(idx_all):
            def grp(g, _):
                idx16 = idx_all[pl.ds(g * L, L)]
                plsc.addupdate_scatter(hist_v, [idx16], ones)
                return 0

            lax.fori_loop(0, eh // L, grp, 0)

        def reduce_dinv(out_v):
            for k in range(NS):
                pltpu.async_copy(
                    sh_hist.at[pl.ds(k * npad + sid * seg, seg)],
                    red_v.at[pl.ds(k * seg, seg)],
                    sem_a,
                )
            for k in range(NS):
                pltpu.make_async_copy(
                    sh_hist.at[pl.ds(k * npad + sid * seg, seg)],
                    red_v.at[pl.ds(k * seg, seg)],
                    sem_a,
                ).wait()

            def red(i, _):
                acc = red_v[pl.ds(i * L, L)]
                for k in range(1, NS):
                    acc = acc + red_v[pl.ds(k * seg + i * L, L)]
                out_v[pl.ds(i * L, L)] = _rsqrt16(acc)
                return 0

            lax.fori_loop(0, seg // L, red, 0)

        lax.fori_loop(0, npad // L, zero_hist, 0)
        pltpu.make_async_copy(
            src_hbm.at[pl.ds(sid * eh, eh)], sidx_all, sem_a
        ).wait()
        accumulate(sidx_all)
        pltpu.sync_copy(hist_v, sh_hist.at[pl.ds(sid * npad, npad)])
        plsc.subcore_barrier()
        reduce_dinv(dvt_v)
        pltpu.sync_copy(dvt_v, sh_dinv.at[pl.ds(sid * seg, seg)])
        plsc.subcore_barrier()
        pltpu.sync_copy(sh_dinv, dinv_v)

        def y_in(jj, b):
            j = jj * NW + wid

            @pl.when(j < nchy)
            def _():
                pltpu.async_copy(
                    x_hbm.at[pl.ds(j * rch, rch), :], xbufs[b], sem_yi.at[b]
                )

        def y_wait_out(jj, b):
            j = jj * NW + wid

            @pl.when(j < nchy)
            def _():
                pltpu.make_async_copy(
                    xbufs[b], y_hbm.at[pl.ds(j * rch, rch), :], sem_yo.at[b]
                ).wait()

        y_in(0, 0)
        for jj in range(njj):
            b = jj % 2
            if jj >= 2:
                y_wait_out(jj - 2, b)
            if jj + 1 < njj:
                y_in(jj + 1, 1 - b)
            j = jj * NW + wid

            @pl.when(j < nchy)
            def _():
                row0 = j * rch
                pltpu.make_async_copy(
                    x_hbm.at[pl.ds(row0, rch), :], xbufs[b], sem_yi.at[b]
                ).wait()

                def row(r, _):
                    rv = jnp.full((L,), r, jnp.int32)
                    wv = plsc.load_gather(
                        dinv_v, [jnp.full((L,), row0 + r, jnp.int32)]
                    )
                    for c in range(d // L):
                        cv = lax.iota(jnp.int32, L) + (c * L)
                        v = plsc.load_gather(xbufs[b], [rv, cv])
                        plsc.store_scatter(xbufs[b], [rv, cv], v * wv)
                    return 0

                lax.fori_loop(0, rch, row, 0)
                pltpu.async_copy(
                    xbufs[b], y_hbm.at[pl.ds(row0, rch), :], sem_yo.at[b]
                )

        for jj in (njj - 2, njj - 1):
            if jj >= 0:
                y_wait_out(jj, jj % 2)

    return k1


def _make_k2(n, d, e, npad, ce, nb=4):
    seg = npad // NS
    ept = e // NW
    nck = ept // ce
    nzc = n // ce
    nzpt = (nzc + NS - 1) // NS
    nslot = nck + nb - 1
    ng = (nslot + nb - 1) // nb
    mesh = plsc.VectorSubcoreMesh(
        core_axis_name="c", subcore_axis_name="s", num_cores=NC, num_subcores=NS
    )

    scratch = (
        [pltpu.VMEM((ce,), jnp.int32) for _ in range(nb)]
        + [pltpu.VMEM((ce,), jnp.int32) for _ in range(nb)]
        + [pltpu.VMEM((ce, d), jnp.float32) for _ in range(nb)]
        + [
            pltpu.VMEM((ce,), jnp.float32),
            pltpu.VMEM((seg,), jnp.float32),
            pltpu.VMEM_SHARED((n, d), jnp.float32),
            pltpu.VMEM_SHARED((npad,), jnp.float32),
            pltpu.SemaphoreType.DMA((nb,)),
            pltpu.SemaphoreType.DMA((nb,)),
            pltpu.SemaphoreType.DMA((nb,)),
            pltpu.SemaphoreType.DMA((nb,)),
            pltpu.SemaphoreType.DMA,
        ]
    )

    @functools.partial(
        pl.kernel,
        out_type=(
            jax.ShapeDtypeStruct((n, d), jnp.float32),
            jax.ShapeDtypeStruct((n, d), jnp.float32),
            jax.ShapeDtypeStruct((npad,), jnp.float32),
            jax.ShapeDtypeStruct((npad,), jnp.float32),
        ),
        mesh=mesh,
        compiler_params=pltpu.CompilerParams(needs_layout_passes=False),
        scratch_types=tuple(scratch),
    )
    def k2(y_hbm, src_hbm, tgt_hbm, z_hbm, p0_hbm, p1_hbm, ht0_hbm, ht1_hbm,
           *scr):
        sidx = scr[:nb]
        tidx = scr[nb:2 * nb]
        rows = scr[2 * nb:3 * nb]
        (ones_v, zseg, acc_sh, sht,
         sem_i, sem_g, sem_s, sem_t, sem_z) = scr[3 * nb:]
        cid = lax.axis_index("c")
        sid = lax.axis_index("s")
        wid = cid * NS + sid
        ones = jnp.ones((L,), jnp.float32)
        zeros = jnp.zeros((L,), jnp.float32)

        def fill_ones(i, _):
            ones_v[pl.ds(i * L, L)] = ones
            return 0

        lax.fori_loop(0, ce // L, fill_ones, 0)

        def fill_zseg(i, _):
            zseg[pl.ds(i * L, L)] = zeros
            return 0

        lax.fori_loop(0, seg // L, fill_zseg, 0)
        pltpu.sync_copy(zseg, sht.at[pl.ds(sid * seg, seg)])

        def zfire(jj, _):
            j = jj * NS + sid

            @pl.when(j < nzc)
            def _():
                pltpu.async_copy(
                    z_hbm, acc_sh.at[pl.ds(j * ce, ce), :], sem_z
                )

            return 0

        lax.fori_loop(0, nzpt, zfire, 0)

        def zdrain(jj, _):
            j = jj * NS + sid

            @pl.when(j < nzc)
            def _():
                pltpu.make_async_copy(
                    z_hbm, acc_sh.at[pl.ds(j * ce, ce), :], sem_z
                ).wait()

            return 0

        lax.fori_loop(0, nzpt, zdrain, 0)
        plsc.subcore_barrier()

        base = wid * ept

        def slot(i, b):
            @pl.when((i >= nb) & (i < nck + nb))
            def _():
                pltpu.make_async_copy(
                    rows[b], acc_sh.at[tidx[b]], sem_s.at[b]
                ).wait()
                pltpu.make_async_copy(
                    ones_v, sht.at[tidx[b]], sem_t.at[b]
                ).wait()

            @pl.when(i < nck)
            def _():
                off = base + i * ce
                pltpu.async_copy(
                    src_hbm.at[pl.ds(off, ce)], sidx[b], sem_i.at[b]
                )
                pltpu.async_copy(
                    tgt_hbm.at[pl.ds(off, ce)], tidx[b], sem_i.at[b]
                )

            j = i - 1
            bj = (b - 1) % nb

            @pl.when((j >= 0) & (j < nck))
            def _():
                pltpu.make_async_copy(
                    src_hbm.at[pl.ds(0, ce)], sidx[bj], sem_i.at[bj]
                ).wait()
                pltpu.make_async_copy(
                    tgt_hbm.at[pl.ds(0, ce)], tidx[bj], sem_i.at[bj]
                ).wait()
                pltpu.async_copy(
                    y_hbm.at[sidx[bj]], rows[bj], sem_g.at[bj]
                )

            k = i - 3
            bk = (b - 3) % nb

            @pl.when((k >= 0) & (k < nck))
            def _():
                pltpu.make_async_copy(
                    y_hbm.at[sidx[bk]], rows[bk], sem_g.at[bk]
                ).wait()
                pltpu.async_copy(
                    rows[bk], acc_sh.at[tidx[bk]], sem_s.at[bk], add=True
                )
                pltpu.async_copy(
                    ones_v, sht.at[tidx[bk]], sem_t.at[bk], add=True
                )

        def outer(g, _):
            for b in range(nb):
                slot(g * nb + b, b)
            return 0

        lax.fori_loop(0, ng, outer, 0)
        plsc.subcore_barrier()

        @pl.when(cid == 0)
        def _():
            pltpu.sync_copy(
                sht.at[pl.ds(sid * seg, seg)],
                ht0_hbm.at[pl.ds(sid * seg, seg)],
            )

        @pl.when(cid == 1)
        def _():
            pltpu.sync_copy(
                sht.at[pl.ds(sid * seg, seg)],
                ht1_hbm.at[pl.ds(sid * seg, seg)],
            )

        def dump(p_hbm):
            def dfire(jj, _):
                j = jj * NS + sid

                @pl.when(j < nzc)
                def _():
                    pltpu.async_copy(
                        acc_sh.at[pl.ds(j * ce, ce), :],
                        p_hbm.at[pl.ds(j * ce, ce), :],
                        sem_z,
                    )

                return 0

            lax.fori_loop(0, nzpt, dfire, 0)

            def ddrain(jj, _):
                j = jj * NS + sid

                @pl.when(j < nzc)
                def _():
                    pltpu.make_async_copy(
                        acc_sh.at[pl.ds(j * ce, ce), :],
                        p_hbm.at[pl.ds(j * ce, ce), :],
                        sem_z,
                    ).wait()

                return 0

            lax.fori_loop(0, nzpt, ddrain, 0)

        @pl.when(cid == 0)
        def _():
            dump(p0_hbm)

        @pl.when(cid == 1)
        def _():
            dump(p1_hbm)

    return k2


def _k3_body(p0_ref, p1_ref, h0_ref, h1_ref, o_ref):
    deg = h0_ref[...] + h1_ref[...]
    dinv = jnp.where(deg > 0.0, jax.lax.rsqrt(deg), 0.0)
    o_ref[...] = (p0_ref[...] + p1_ref[...]) * dinv


def kernel(x, edge_index, num_nodes_target):
    n, d = x.shape
    e = edge_index.shape[1]
    npad = ((n + NS * L - 1) // (NS * L)) * (NS * L)
    src = edge_index[0]
    tgt = edge_index[1]

    y = _make_k1(n, d, e, npad, rch=80)(x, src)
    zeros2d = jnp.zeros((80, d), jnp.float32)
    p0, p1, ht0, ht1 = _make_k2(n, d, e, npad, ce=80)(y, src, tgt, zeros2d)

    br = 400
    out = pl.pallas_call(
        _k3_body,
        grid=(n // br,),
        in_specs=[
            pl.BlockSpec((br, d), lambda i: (i, 0)),
            pl.BlockSpec((br, d), lambda i: (i, 0)),
            pl.BlockSpec((br, 1), lambda i: (i, 0)),
            pl.BlockSpec((br, 1), lambda i: (i, 0)),
        ],
        out_specs=pl.BlockSpec((br, d), lambda i: (i, 0)),
        out_shape=jax.ShapeDtypeStruct((n, d), jnp.float32),
    )(p0, p1, ht0.reshape(npad, 1), ht1.reshape(npad, 1))
    return out
